# Initial kernel scaffold; baseline (speedup 1.0000x reference)
import jax
import jax.numpy as jnp
from jax.experimental import pallas as pl


def kernel(input, value):
    raise NotImplementedError("write your pallas kernel here")


# speedup vs baseline: 400.1511x; 400.1511x over previous
import functools

import jax
import jax.numpy as jnp
from jax import lax
from jax.experimental import pallas as pl
from jax.experimental.pallas import tpu as pltpu
from jax.experimental.pallas import tpu_sc as plsc

_N = 100000
_H = 1.0 / (_N - 1)
_EPS = 1e-9
_NE_PAD = _N + 8
_TOTAL = 16384 * 200
_NW = 32
_PER_W = _TOTAL // _NW
_CHUNK = 4096
_NCHUNK = _PER_W // _CHUNK
_L = 16
_NVEC = _CHUNK // _L


def _sc_body(inp_hbm, ye_hbm, out_hbm, ye_v, in_v, out_v):
    nc = 2
    wid = lax.axis_index("s") * nc + lax.axis_index("c")
    pltpu.sync_copy(ye_hbm, ye_v)
    base = wid * _PER_W
    hf = jnp.float32(_H)

    def chunk_body(c, carry):
        off = base + c * _CHUNK
        pltpu.sync_copy(inp_hbm.at[pl.ds(off, _CHUNK)], in_v)

        def vec_body(j, carry2):
            x = in_v[pl.ds(j * _L, _L)]
            uc = jnp.minimum(jnp.maximum(x, jnp.float32(0.0)), jnp.float32(1.0))
            pos = uc / hf
            idx = jnp.minimum(pos.astype(jnp.int32), _N - 2)
            s = pos - idx.astype(jnp.float32)
            s = jnp.minimum(jnp.maximum(s, jnp.float32(0.0)), jnp.float32(1.0))
            g0 = plsc.load_gather(ye_v, [idx])
            g1 = plsc.load_gather(ye_v, [idx + 1])
            g2 = plsc.load_gather(ye_v, [idx + 2])
            g3 = plsc.load_gather(ye_v, [idx + 3])
            g4 = plsc.load_gather(ye_v, [idx + 4])
            g5 = plsc.load_gather(ye_v, [idx + 5])
            m0 = (g1 - g0) / hf
            m1 = (g2 - g1) / hf
            m2 = (g3 - g2) / hf
            m3 = (g4 - g3) / hf
            m4 = (g5 - g4) / hf
            w10 = jnp.abs(m3 - m2)
            w20 = jnp.abs(m1 - m0)
            den0 = w10 + w20
            p0 = den0 > jnp.float32(_EPS)
            t0 = jnp.where(
                p0,
                (w10 * m1 + w20 * m2) / jnp.where(p0, den0, jnp.float32(1.0)),
                jnp.float32(0.5) * (m1 + m2),
            )
            w11 = jnp.abs(m4 - m3)
            w21 = jnp.abs(m2 - m1)
            den1 = w11 + w21
            p1 = den1 > jnp.float32(_EPS)
            t1 = jnp.where(
                p1,
                (w11 * m2 + w21 * m3) / jnp.where(p1, den1, jnp.float32(1.0)),
                jnp.float32(0.5) * (m2 + m3),
            )
            s2 = s * s
            s3 = s2 * s
            h00 = jnp.float32(2.0) * s3 - jnp.float32(3.0) * s2 + jnp.float32(1.0)
            h10 = s3 - jnp.float32(2.0) * s2 + s
            h01 = jnp.float32(-2.0) * s3 + jnp.float32(3.0) * s2
            h11 = s3 - s2
            r = h00 * g2 + h10 * (hf * t0) + h01 * g3 + h11 * (hf * t1)
            out_v[pl.ds(j * _L, _L)] = r
            return carry2

        lax.fori_loop(0, _NVEC, vec_body, 0)
        pltpu.sync_copy(out_v, out_hbm.at[pl.ds(off, _CHUNK)])
        return carry

    lax.fori_loop(0, _NCHUNK, chunk_body, 0)


_sc_interp = functools.partial(
    pl.kernel,
    out_type=jax.ShapeDtypeStruct((_TOTAL,), jnp.float32),
    mesh=plsc.VectorSubcoreMesh(core_axis_name="c", subcore_axis_name="s"),
    compiler_params=pltpu.CompilerParams(needs_layout_passes=False),
    scratch_types=[
        pltpu.VMEM((_NE_PAD,), jnp.float32),
        pltpu.VMEM((_CHUNK,), jnp.float32),
        pltpu.VMEM((_CHUNK,), jnp.float32),
    ],
)(_sc_body)


def kernel(input, value):
    y = value
    h = _H
    m_a = (y[1] - y[0]) / h
    m_b = (y[2] - y[1]) / h
    mm1 = 2.0 * m_a - m_b
    mm2 = 2.0 * mm1 - m_a
    m_y = (y[-1] - y[-2]) / h
    m_x = (y[-2] - y[-3]) / h
    mp1 = 2.0 * m_y - m_x
    mp2 = 2.0 * mp1 - m_y
    ym1 = y[0] - h * mm1
    ym2 = ym1 - h * mm2
    yp1 = y[-1] + h * mp1
    yp2 = yp1 + h * mp2
    ye = jnp.concatenate([
        jnp.stack([ym2, ym1]),
        y,
        jnp.stack([yp1, yp2]),
        jnp.zeros((4,), jnp.float32),
    ])
    flat = input.reshape(-1)
    out = _sc_interp(flat, ye)
    return out.reshape(input.shape)


# speedup vs baseline: 451.9035x; 1.1293x over previous
import functools

import jax
import jax.numpy as jnp
from jax import lax
from jax.experimental import pallas as pl
from jax.experimental.pallas import tpu as pltpu
from jax.experimental.pallas import tpu_sc as plsc

_N = 100000
_H = 1.0 / (_N - 1)
_NE_PAD = _N + 8
_TOTAL = 16384 * 200
_NW = 32
_PER_W = _TOTAL // _NW
_CHUNK = 6400
_NCHUNK = _PER_W // _CHUNK
_NPAIR = _NCHUNK // 2
_L = 16
_NVEC = _CHUNK // _L


def _sc_body(inp_hbm, ye_hbm, out_hbm, ye_v,
             in0_v, in1_v, out0_v, out1_v,
             sem_i0, sem_i1, sem_o0, sem_o1):
    nc = 2
    wid = lax.axis_index("s") * nc + lax.axis_index("c")
    pltpu.sync_copy(ye_hbm, ye_v)
    base = wid * _PER_W
    hf = jnp.float32(_H)
    epsd = jnp.float32(1e-9 * _H)
    in_bufs = (in0_v, in1_v)
    out_bufs = (out0_v, out1_v)
    sems_i = (sem_i0, sem_i1)
    sems_o = (sem_o0, sem_o1)

    def in_slice(c):
        return inp_hbm.at[pl.ds(base + c * _CHUNK, _CHUNK)]

    def out_slice(c):
        return out_hbm.at[pl.ds(base + c * _CHUNK, _CHUNK)]

    def compute(in_v, out_v):
        def vec_body(j, carry):
            x = in_v[pl.ds(j * _L, _L)]
            uc = jnp.minimum(jnp.maximum(x, jnp.float32(0.0)), jnp.float32(1.0))
            pos = uc / hf
            idx = jnp.minimum(pos.astype(jnp.int32), _N - 2)
            s = pos - idx.astype(jnp.float32)
            s = jnp.minimum(jnp.maximum(s, jnp.float32(0.0)), jnp.float32(1.0))
            g0 = plsc.load_gather(ye_v, [idx])
            g1 = plsc.load_gather(ye_v, [idx + 1])
            g2 = plsc.load_gather(ye_v, [idx + 2])
            g3 = plsc.load_gather(ye_v, [idx + 3])
            g4 = plsc.load_gather(ye_v, [idx + 4])
            g5 = plsc.load_gather(ye_v, [idx + 5])
            d0 = g1 - g0
            d1 = g2 - g1
            d2 = g3 - g2
            d3 = g4 - g3
            d4 = g5 - g4
            w10 = jnp.abs(d3 - d2)
            w20 = jnp.abs(d1 - d0)
            den0 = w10 + w20
            p0 = den0 > epsd
            t0h = jnp.where(
                p0,
                (w10 * d1 + w20 * d2) / jnp.where(p0, den0, jnp.float32(1.0)),
                jnp.float32(0.5) * (d1 + d2),
            )
            w11 = jnp.abs(d4 - d3)
            w21 = jnp.abs(d2 - d1)
            den1 = w11 + w21
            p1 = den1 > epsd
            t1h = jnp.where(
                p1,
                (w11 * d2 + w21 * d3) / jnp.where(p1, den1, jnp.float32(1.0)),
                jnp.float32(0.5) * (d2 + d3),
            )
            s2 = s * s
            s3 = s2 * s
            h00 = jnp.float32(2.0) * s3 - jnp.float32(3.0) * s2 + jnp.float32(1.0)
            h10 = s3 - jnp.float32(2.0) * s2 + s
            h01 = jnp.float32(3.0) * s2 - jnp.float32(2.0) * s3
            h11 = s3 - s2
            r = h00 * g2 + h10 * t0h + h01 * g3 + h11 * t1h
            out_v[pl.ds(j * _L, _L)] = r
            return carry

        lax.fori_loop(0, _NVEC, vec_body, 0)

    pltpu.async_copy(in_slice(0), in0_v, sem_i0)
    pltpu.async_copy(in_slice(1), in1_v, sem_i1)

    for b in range(2):
        pltpu.make_async_copy(in_slice(b), in_bufs[b], sems_i[b]).wait()
        compute(in_bufs[b], out_bufs[b])
        pltpu.async_copy(in_slice(b + 2), in_bufs[b], sems_i[b])
        pltpu.async_copy(out_bufs[b], out_slice(b), sems_o[b])

    def pair_body(i, carry):
        c0 = 2 * i
        for b in range(2):
            c = c0 + b
            pltpu.make_async_copy(in_slice(c), in_bufs[b], sems_i[b]).wait()
            pltpu.make_async_copy(out_bufs[b], out_slice(c - 2), sems_o[b]).wait()
            compute(in_bufs[b], out_bufs[b])
            pltpu.async_copy(in_slice(c + 2), in_bufs[b], sems_i[b])
            pltpu.async_copy(out_bufs[b], out_slice(c), sems_o[b])
        return carry

    lax.fori_loop(1, _NPAIR - 1, pair_body, 0)

    for b in range(2):
        c = _NCHUNK - 2 + b
        pltpu.make_async_copy(in_slice(c), in_bufs[b], sems_i[b]).wait()
        pltpu.make_async_copy(out_bufs[b], out_slice(c - 2), sems_o[b]).wait()
        compute(in_bufs[b], out_bufs[b])
        pltpu.async_copy(out_bufs[b], out_slice(c), sems_o[b])

    for b in range(2):
        c = _NCHUNK - 2 + b
        pltpu.make_async_copy(out_bufs[b], out_slice(c), sems_o[b]).wait()


_sc_interp = functools.partial(
    pl.kernel,
    out_type=jax.ShapeDtypeStruct((_TOTAL,), jnp.float32),
    mesh=plsc.VectorSubcoreMesh(core_axis_name="c", subcore_axis_name="s"),
    compiler_params=pltpu.CompilerParams(needs_layout_passes=False),
    scratch_types=[
        pltpu.VMEM((_NE_PAD,), jnp.float32),
        pltpu.VMEM((_CHUNK,), jnp.float32),
        pltpu.VMEM((_CHUNK,), jnp.float32),
        pltpu.VMEM((_CHUNK,), jnp.float32),
        pltpu.VMEM((_CHUNK,), jnp.float32),
        pltpu.SemaphoreType.DMA,
        pltpu.SemaphoreType.DMA,
        pltpu.SemaphoreType.DMA,
        pltpu.SemaphoreType.DMA,
    ],
)(_sc_body)


def kernel(input, value):
    y = value
    h = _H
    m_a = (y[1] - y[0]) / h
    m_b = (y[2] - y[1]) / h
    mm1 = 2.0 * m_a - m_b
    mm2 = 2.0 * mm1 - m_a
    m_y = (y[-1] - y[-2]) / h
    m_x = (y[-2] - y[-3]) / h
    mp1 = 2.0 * m_y - m_x
    mp2 = 2.0 * mp1 - m_y
    ym1 = y[0] - h * mm1
    ym2 = ym1 - h * mm2
    yp1 = y[-1] + h * mp1
    yp2 = yp1 + h * mp2
    ye = jnp.concatenate([
        jnp.stack([ym2, ym1]),
        y,
        jnp.stack([yp1, yp2]),
        jnp.zeros((4,), jnp.float32),
    ])
    flat = input.reshape(-1)
    out = _sc_interp(flat, ye)
    return out.reshape(input.shape)
